# baseline (speedup 1.0000x reference)
import functools

import jax
import jax.numpy as jnp
from jax.experimental import pallas as pl
from jax.experimental.pallas import tpu as pltpu


def _se_kernel(x_ref, w1_ref, b1_ref, w2_ref, b2_ref, o_ref, *, inv_hw):
    x = x_ref[...]
    s = jnp.sum(x, axis=(2, 3)) * inv_hw
    z = jnp.dot(s, w1_ref[...], preferred_element_type=jnp.float32)
    z = z + b1_ref[...]
    z = jnp.where(z > 0, z, jnp.exp(jnp.minimum(z, 0.0)) - 1.0)
    g = jnp.dot(z, w2_ref[...], preferred_element_type=jnp.float32)
    g = jax.nn.sigmoid(g + b2_ref[...])
    o_ref[...] = x * g[:, :, None, None]


def kernel(x_nchw, w1, b1, w2, b2):
    B, C, H, W = x_nchw.shape
    Cr = w1.shape[1]
    HH, WW = (H * W) // 128, 128
    xv = x_nchw.reshape(B, C, HH, WW)

    b1r = b1.reshape(1, Cr).astype(jnp.float32)
    b2r = b2.reshape(1, C).astype(jnp.float32)
    w1f = w1.astype(jnp.float32)
    w2f = w2.astype(jnp.float32)

    itemsize = jnp.dtype(x_nchw.dtype).itemsize
    out = pl.pallas_call(
        functools.partial(_se_kernel, inv_hw=1.0 / float(H * W)),
        out_shape=jax.ShapeDtypeStruct((B, C, HH, WW), x_nchw.dtype),
        grid=(B,),
        in_specs=[
            pl.BlockSpec((1, C, HH, WW), lambda b: (b, 0, 0, 0)),
            pl.BlockSpec((C, Cr), lambda b: (0, 0)),
            pl.BlockSpec((1, Cr), lambda b: (0, 0)),
            pl.BlockSpec((Cr, C), lambda b: (0, 0)),
            pl.BlockSpec((1, C), lambda b: (0, 0)),
        ],
        out_specs=pl.BlockSpec((1, C, HH, WW), lambda b: (b, 0, 0, 0)),
        compiler_params=pltpu.CompilerParams(
            dimension_semantics=("parallel",),
            vmem_limit_bytes=48 * 1024 * 1024,
        ),
        cost_estimate=pl.CostEstimate(
            flops=2 * B * C * H * W + 4 * B * C * Cr,
            transcendentals=B * C + B * Cr,
            bytes_accessed=2 * x_nchw.size * itemsize,
        ),
    )(xv, w1f, b1r, w2f, b2r)

    return out.reshape(B, C, H, W)


# speedup vs baseline: 3.8683x; 3.8683x over previous
import functools

import jax
import jax.numpy as jnp
from jax.experimental import pallas as pl
from jax.experimental.pallas import tpu as pltpu


def _se_kernel(x_ref, w1_ref, b1_ref, w2_ref, b2_ref, o_ref, *, inv_hw):
    x = x_ref[...]
    s = jnp.sum(x, axis=(1, 2)) * inv_hw
    z = jnp.dot(s, w1_ref[...], preferred_element_type=jnp.float32)
    z = z + b1_ref[...]
    z = jnp.where(z > 0, z, jnp.exp(jnp.minimum(z, 0.0)) - 1.0)
    g = jnp.dot(z, w2_ref[...], preferred_element_type=jnp.float32)
    g = jax.nn.sigmoid(g + b2_ref[...])
    o_ref[...] = x * g[:, None, None, :]


def kernel(x_nchw, w1, b1, w2, b2):
    B, C, H, W = x_nchw.shape
    Cr = w1.shape[1]

    x_nhwc = x_nchw.transpose(0, 2, 3, 1)

    b1r = b1.reshape(1, Cr).astype(jnp.float32)
    b2r = b2.reshape(1, C).astype(jnp.float32)
    w1f = w1.astype(jnp.float32)
    w2f = w2.astype(jnp.float32)

    itemsize = jnp.dtype(x_nchw.dtype).itemsize
    out = pl.pallas_call(
        functools.partial(_se_kernel, inv_hw=1.0 / float(H * W)),
        out_shape=jax.ShapeDtypeStruct((B, H, W, C), x_nchw.dtype),
        grid=(B,),
        in_specs=[
            pl.BlockSpec((1, H, W, C), lambda b: (b, 0, 0, 0)),
            pl.BlockSpec((C, Cr), lambda b: (0, 0)),
            pl.BlockSpec((1, Cr), lambda b: (0, 0)),
            pl.BlockSpec((Cr, C), lambda b: (0, 0)),
            pl.BlockSpec((1, C), lambda b: (0, 0)),
        ],
        out_specs=pl.BlockSpec((1, H, W, C), lambda b: (b, 0, 0, 0)),
        compiler_params=pltpu.CompilerParams(
            dimension_semantics=("parallel",),
            vmem_limit_bytes=48 * 1024 * 1024,
        ),
        cost_estimate=pl.CostEstimate(
            flops=2 * B * C * H * W + 4 * B * C * Cr,
            transcendentals=B * C + B * Cr,
            bytes_accessed=2 * x_nchw.size * itemsize,
        ),
    )(x_nhwc, w1f, b1r, w2f, b2r)

    return out.transpose(0, 3, 1, 2)
